# baseline (speedup 1.0000x reference)
import functools

import jax
import jax.numpy as jnp
from jax import lax
from jax.experimental import pallas as pl
from jax.experimental.pallas import tpu as pltpu
from jax.experimental.pallas import tpu_sc as plsc

N = 10000
E = 320000
D = 128
H = 128
C = 64

NC = 2
NS = 16
NW = NC * NS
LANES = 16

CHUNK = 128
SPLIT = 2
SUB = CHUNK // SPLIT
K = 80
KH = K // 2
EPW = K * CHUNK
NPAD = 10240
ROWS_PER_TILE = NPAD // NS
DCHUNK = 128
DK = EPW // DCHUNK
RB = 1000


def _agg_body(h_hbm, src_hbm, dst_hbm, out_hbm,
              acc, src_v, dst_v, rows0, rows1,
              sem00, sem01, sem10, sem11):
    c = lax.axis_index("c")
    s = lax.axis_index("s")
    wid = s * NC + c
    rows = [rows0, rows1]
    sems = [[sem00, sem01], [sem10, sem11]]

    zero16 = jnp.zeros((LANES,), jnp.float32)

    def _zrow(i, _):
        for l in range(H // LANES):
            rows0[i, pl.ds(l * LANES, LANES)] = zero16
        return 0
    lax.fori_loop(0, CHUNK, _zrow, 0)
    for t in range(ROWS_PER_TILE // CHUNK):
        pltpu.sync_copy(rows0, acc.at[pl.ds(s * ROWS_PER_TILE + t * CHUNK, CHUNK)])

    plsc.subcore_barrier()

    def _issue(j, b):
        for p in range(SPLIT):
            pltpu.async_copy(
                h_hbm.at[src_v.at[j, pl.ds(p * SUB, SUB)]],
                rows[b].at[pl.ds(p * SUB, SUB)], sems[b][p])

    def _wait(b):
        for p in range(SPLIT):
            pltpu.make_async_copy(
                h_hbm.at[src_v.at[0, pl.ds(0, SUB)]],
                rows[b].at[pl.ds(0, SUB)], sems[b][p]).wait()

    for ph in range(2):
        pltpu.sync_copy(src_hbm.at[wid, ph], src_v)
        pltpu.sync_copy(dst_hbm.at[wid, ph], dst_v)

        _issue(0, 0)
        _issue(1, 1)

        def _edge_chunk(j, b):
            _wait(b)
            pltpu.sync_copy(rows[b], acc.at[dst_v.at[j]], add=True)
            _issue(j + 2, b)

        def _main(jj, _):
            _edge_chunk(2 * jj, 0)
            _edge_chunk(2 * jj + 1, 1)
            return 0
        lax.fori_loop(0, KH // 2, _main, 0)

        _wait(0)
        _wait(1)

    plsc.subcore_barrier()
    pltpu.sync_copy(acc.at[pl.ds(s * ROWS_PER_TILE, ROWS_PER_TILE)],
                    out_hbm.at[c, pl.ds(s * ROWS_PER_TILE, ROWS_PER_TILE)])


_sc_agg = pl.kernel(
    _agg_body,
    out_type=[jax.ShapeDtypeStruct((NC, NPAD, H), jnp.float32)],
    mesh=plsc.VectorSubcoreMesh(core_axis_name="c", subcore_axis_name="s"),
    scratch_types=[
        pltpu.VMEM_SHARED((NPAD, H), jnp.float32),
        pltpu.VMEM((KH + 2, CHUNK), jnp.int32),
        pltpu.VMEM((KH, CHUNK), jnp.int32),
        pltpu.VMEM((CHUNK, H), jnp.float32),
        pltpu.VMEM((CHUNK, H), jnp.float32),
        pltpu.SemaphoreType.DMA,
        pltpu.SemaphoreType.DMA,
        pltpu.SemaphoreType.DMA,
        pltpu.SemaphoreType.DMA,
    ],
)


def _deg_body(dst_hbm, deg_hbm, degacc, dst_v, onesbuf, sem):
    c = lax.axis_index("c")
    s = lax.axis_index("s")
    wid = s * NC + c

    zero16 = jnp.zeros((LANES,), jnp.float32)
    ones16 = jnp.ones((LANES,), jnp.float32)

    def _zd(i, _):
        onesbuf[i, :] = zero16
        return 0
    lax.fori_loop(0, DCHUNK, _zd, 0)
    for t in range(ROWS_PER_TILE // DCHUNK):
        pltpu.sync_copy(
            onesbuf, degacc.at[pl.ds(s * ROWS_PER_TILE + t * DCHUNK, DCHUNK)])

    def _od(i, _):
        onesbuf[i, :] = ones16
        return 0
    lax.fori_loop(0, DCHUNK, _od, 0)

    pltpu.sync_copy(dst_hbm.at[wid], dst_v)
    plsc.subcore_barrier()

    def _body(j, _):
        pltpu.sync_copy(onesbuf, degacc.at[dst_v.at[j]], add=True)
        return 0
    lax.fori_loop(0, DK, _body, 0)

    plsc.subcore_barrier()
    pltpu.sync_copy(degacc.at[pl.ds(s * ROWS_PER_TILE, ROWS_PER_TILE)],
                    deg_hbm.at[c, pl.ds(s * ROWS_PER_TILE, ROWS_PER_TILE)])


_sc_deg = pl.kernel(
    _deg_body,
    out_type=[jax.ShapeDtypeStruct((NC, NPAD, LANES), jnp.float32)],
    mesh=plsc.VectorSubcoreMesh(core_axis_name="c", subcore_axis_name="s"),
    scratch_types=[
        pltpu.VMEM_SHARED((NPAD, LANES), jnp.float32),
        pltpu.VMEM((DK, DCHUNK), jnp.int32),
        pltpu.VMEM((DCHUNK, LANES), jnp.float32),
        pltpu.SemaphoreType.DMA,
    ],
    compiler_params=pltpu.CompilerParams(use_tc_tiling_on_sc=False),
)


def _layer_body(relu, x_ref, wsT_ref, b_ref, p0_ref, p1_ref,
                da_ref, db_ref, wnT_ref, o_ref):
    inv = 1.0 / jnp.maximum(da_ref[...] + db_ref[...], 1.0)
    agg = (p0_ref[...] + p1_ref[...]) * inv
    pre = (jnp.dot(x_ref[...], wsT_ref[...], preferred_element_type=jnp.float32)
           + jnp.dot(agg, wnT_ref[...], preferred_element_type=jnp.float32)
           + b_ref[...])
    o_ref[...] = jnp.maximum(pre, 0.0) if relu else pre


def _tc_layer(relu, x, wsT, b, p0, p1, da, db, wnT):
    hd = wsT.shape[1]
    return pl.pallas_call(
        functools.partial(_layer_body, relu),
        grid=(N // RB,),
        in_specs=[pl.BlockSpec((RB, H), lambda i: (i, 0)),
                  pl.BlockSpec((H, hd), lambda i: (0, 0)),
                  pl.BlockSpec((1, hd), lambda i: (0, 0)),
                  pl.BlockSpec((RB, H), lambda i: (i, 0)),
                  pl.BlockSpec((RB, H), lambda i: (i, 0)),
                  pl.BlockSpec((RB, 1), lambda i: (i, 0)),
                  pl.BlockSpec((RB, 1), lambda i: (i, 0)),
                  pl.BlockSpec((H, hd), lambda i: (0, 0))],
        out_specs=pl.BlockSpec((RB, hd), lambda i: (i, 0)),
        out_shape=jax.ShapeDtypeStruct((N, hd), jnp.float32),
    )(x, wsT, b, p0, p1, da, db, wnT)


def kernel(x, edge_index, Ws1, Wn1, b1, Ws2, Wn2, b2, Ws3, Wn3, b3):
    src = edge_index[0]
    dst = edge_index[1]

    pad = NW * EPW - E
    src_p = jnp.concatenate([src, jnp.zeros((pad,), jnp.int32)])
    dum = N + (jnp.arange(pad, dtype=jnp.int32) % (NPAD - N))
    dst_p = jnp.concatenate([dst, dum])
    src_t = jnp.concatenate(
        [src_p.reshape(NW, 2, KH, CHUNK),
         jnp.zeros((NW, 2, 2, CHUNK), jnp.int32)], axis=2)
    dst_t = dst_p.reshape(NW, 2, KH, CHUNK)
    dst_d = dst_p.reshape(NW, DK, DCHUNK)

    ws1T, wn1T = Ws1.T, Wn1.T
    ws2T, wn2T = Ws2.T, Wn2.T
    ws3T, wn3T = Ws3.T, Wn3.T
    b1r = b1.reshape(1, H)
    b2r = b2.reshape(1, H)
    b3r = b3.reshape(1, C)

    degp, = _sc_deg(dst_d)
    da = degp[0, :, 0:1]
    db = degp[1, :, 0:1]

    p1, = _sc_agg(x, src_t, dst_t)
    h1 = _tc_layer(True, x, ws1T, b1r, p1[0], p1[1], da, db, wn1T)
    p2, = _sc_agg(h1, src_t, dst_t)
    h2 = _tc_layer(True, h1, ws2T, b2r, p2[0], p2[1], da, db, wn2T)
    p3, = _sc_agg(h2, src_t, dst_t)
    out = _tc_layer(False, h2, ws3T, b3r, p3[0], p3[1], da, db, wn3T)
    return out


# speedup vs baseline: 2.3220x; 2.3220x over previous
import functools

import jax
import jax.numpy as jnp
from jax import lax
from jax.experimental import pallas as pl
from jax.experimental.pallas import tpu as pltpu
from jax.experimental.pallas import tpu_sc as plsc

N = 10000
E = 320000
D = 128
H = 128
C = 64

NC = 2
NS = 16
NW = NC * NS
LANES = 16

CHUNK = 64
NBUF = 4
K = 160
NPH = 4
KH = K // NPH
EPW = K * CHUNK
NPAD = 10240
ROWS_PER_TILE = NPAD // NS
DCHUNK = 128
DK = EPW // DCHUNK
RB = 1000


def _agg_body(h_hbm, src_hbm, dst_hbm, out_hbm,
              acc, src_v, dst_v, r0, r1, r2, r3,
              g0, g1, g2, g3, s0, s1, s2, s3):
    c = lax.axis_index("c")
    s = lax.axis_index("s")
    wid = s * NC + c
    rows = [r0, r1, r2, r3]
    gsem = [g0, g1, g2, g3]
    ssem = [s0, s1, s2, s3]

    zero16 = jnp.zeros((LANES,), jnp.float32)

    def _zrow(i, _):
        for l in range(H // LANES):
            r0[i, pl.ds(l * LANES, LANES)] = zero16
        return 0
    lax.fori_loop(0, CHUNK, _zrow, 0)
    for t in range(ROWS_PER_TILE // CHUNK):
        pltpu.sync_copy(r0, acc.at[pl.ds(s * ROWS_PER_TILE + t * CHUNK, CHUNK)])

    plsc.subcore_barrier()

    def _ig(j, b):
        pltpu.async_copy(h_hbm.at[src_v.at[j]], rows[b], gsem[b])

    def _wg(b):
        pltpu.make_async_copy(h_hbm.at[src_v.at[0]], rows[b], gsem[b]).wait()

    def _is(j, b):
        pltpu.async_copy(rows[b], acc.at[dst_v.at[j]], ssem[b], add=True)

    def _ws(b):
        pltpu.make_async_copy(h_hbm.at[pl.ds(0, CHUNK)], rows[b], ssem[b]).wait()

    for ph in range(NPH):
        pltpu.sync_copy(src_hbm.at[wid, ph], src_v)
        pltpu.sync_copy(dst_hbm.at[wid, ph], dst_v)

        _ig(0, 0)
        _ig(1, 1)
        _wg(0); _is(0, 0); _ig(2, 2)
        _wg(1); _is(1, 1); _ig(3, 3)

        def _grp(jj, _):
            j0 = 4 * jj + 2
            for t in range(4):
                j = j0 + t
                b = (2 + t) % 4
                bn = t % 4
                _wg(b)
                _is(j, b)
                _ws(bn)
                _ig(j + 2, bn)
            return 0
        lax.fori_loop(0, (KH - 4) // 4, _grp, 0)

        _wg(2); _is(KH - 2, 2)
        _wg(3); _is(KH - 1, 3)
        for b in range(NBUF):
            _ws(b)

    plsc.subcore_barrier()
    pltpu.sync_copy(acc.at[pl.ds(s * ROWS_PER_TILE, ROWS_PER_TILE)],
                    out_hbm.at[c, pl.ds(s * ROWS_PER_TILE, ROWS_PER_TILE)])


_sc_agg = pl.kernel(
    _agg_body,
    out_type=[jax.ShapeDtypeStruct((NC, NPAD, H), jnp.float32)],
    mesh=plsc.VectorSubcoreMesh(core_axis_name="c", subcore_axis_name="s"),
    scratch_types=[
        pltpu.VMEM_SHARED((NPAD, H), jnp.float32),
        pltpu.VMEM((KH, CHUNK), jnp.int32),
        pltpu.VMEM((KH, CHUNK), jnp.int32),
        pltpu.VMEM((CHUNK, H), jnp.float32),
        pltpu.VMEM((CHUNK, H), jnp.float32),
        pltpu.VMEM((CHUNK, H), jnp.float32),
        pltpu.VMEM((CHUNK, H), jnp.float32),
        pltpu.SemaphoreType.DMA,
        pltpu.SemaphoreType.DMA,
        pltpu.SemaphoreType.DMA,
        pltpu.SemaphoreType.DMA,
        pltpu.SemaphoreType.DMA,
        pltpu.SemaphoreType.DMA,
        pltpu.SemaphoreType.DMA,
        pltpu.SemaphoreType.DMA,
    ],
)


def _deg_body(dst_hbm, deg_hbm, degacc, dst_v, onesbuf, sem):
    c = lax.axis_index("c")
    s = lax.axis_index("s")
    wid = s * NC + c

    zero16 = jnp.zeros((LANES,), jnp.float32)
    ones16 = jnp.ones((LANES,), jnp.float32)

    def _zd(i, _):
        onesbuf[i, :] = zero16
        return 0
    lax.fori_loop(0, DCHUNK, _zd, 0)
    for t in range(ROWS_PER_TILE // DCHUNK):
        pltpu.sync_copy(
            onesbuf, degacc.at[pl.ds(s * ROWS_PER_TILE + t * DCHUNK, DCHUNK)])

    def _od(i, _):
        onesbuf[i, :] = ones16
        return 0
    lax.fori_loop(0, DCHUNK, _od, 0)

    pltpu.sync_copy(dst_hbm.at[wid], dst_v)
    plsc.subcore_barrier()

    def _body(j, _):
        pltpu.sync_copy(onesbuf, degacc.at[dst_v.at[j]], add=True)
        return 0
    lax.fori_loop(0, DK, _body, 0)

    plsc.subcore_barrier()
    pltpu.sync_copy(degacc.at[pl.ds(s * ROWS_PER_TILE, ROWS_PER_TILE)],
                    deg_hbm.at[c, pl.ds(s * ROWS_PER_TILE, ROWS_PER_TILE)])


_sc_deg = pl.kernel(
    _deg_body,
    out_type=[jax.ShapeDtypeStruct((NC, NPAD, LANES), jnp.float32)],
    mesh=plsc.VectorSubcoreMesh(core_axis_name="c", subcore_axis_name="s"),
    scratch_types=[
        pltpu.VMEM_SHARED((NPAD, LANES), jnp.float32),
        pltpu.VMEM((DK, DCHUNK), jnp.int32),
        pltpu.VMEM((DCHUNK, LANES), jnp.float32),
        pltpu.SemaphoreType.DMA,
    ],
    compiler_params=pltpu.CompilerParams(use_tc_tiling_on_sc=False),
)


def _layer_body(relu, x_ref, wsT_ref, b_ref, p0_ref, p1_ref,
                da_ref, db_ref, wnT_ref, o_ref):
    inv = 1.0 / jnp.maximum(da_ref[...] + db_ref[...], 1.0)
    agg = (p0_ref[...] + p1_ref[...]) * inv
    pre = (jnp.dot(x_ref[...], wsT_ref[...], preferred_element_type=jnp.float32)
           + jnp.dot(agg, wnT_ref[...], preferred_element_type=jnp.float32)
           + b_ref[...])
    o_ref[...] = jnp.maximum(pre, 0.0) if relu else pre


def _tc_layer(relu, x, wsT, b, p0, p1, da, db, wnT):
    hd = wsT.shape[1]
    return pl.pallas_call(
        functools.partial(_layer_body, relu),
        grid=(N // RB,),
        in_specs=[pl.BlockSpec((RB, H), lambda i: (i, 0)),
                  pl.BlockSpec((H, hd), lambda i: (0, 0)),
                  pl.BlockSpec((1, hd), lambda i: (0, 0)),
                  pl.BlockSpec((RB, H), lambda i: (i, 0)),
                  pl.BlockSpec((RB, H), lambda i: (i, 0)),
                  pl.BlockSpec((RB, 1), lambda i: (i, 0)),
                  pl.BlockSpec((RB, 1), lambda i: (i, 0)),
                  pl.BlockSpec((H, hd), lambda i: (0, 0))],
        out_specs=pl.BlockSpec((RB, hd), lambda i: (i, 0)),
        out_shape=jax.ShapeDtypeStruct((N, hd), jnp.float32),
    )(x, wsT, b, p0, p1, da, db, wnT)


def kernel(x, edge_index, Ws1, Wn1, b1, Ws2, Wn2, b2, Ws3, Wn3, b3):
    src = edge_index[0]
    dst = edge_index[1]

    pad = NW * EPW - E
    src_p = jnp.concatenate([src, jnp.zeros((pad,), jnp.int32)])
    dum = N + (jnp.arange(pad, dtype=jnp.int32) % (NPAD - N))
    dst_p = jnp.concatenate([dst, dum])
    src_t = src_p.reshape(NW, NPH, KH, CHUNK)
    dst_t = dst_p.reshape(NW, NPH, KH, CHUNK)
    dst_d = dst_p.reshape(NW, DK, DCHUNK)

    ws1T, wn1T = Ws1.T, Wn1.T
    ws2T, wn2T = Ws2.T, Wn2.T
    ws3T, wn3T = Ws3.T, Wn3.T
    b1r = b1.reshape(1, H)
    b2r = b2.reshape(1, H)
    b3r = b3.reshape(1, C)

    degp, = _sc_deg(dst_d)
    da = degp[0, :, 0:1]
    db = degp[1, :, 0:1]

    p1, = _sc_agg(x, src_t, dst_t)
    h1 = _tc_layer(True, x, ws1T, b1r, p1[0], p1[1], da, db, wn1T)
    p2, = _sc_agg(h1, src_t, dst_t)
    h2 = _tc_layer(True, h1, ws2T, b2r, p2[0], p2[1], da, db, wn2T)
    p3, = _sc_agg(h2, src_t, dst_t)
    out = _tc_layer(False, h2, ws3T, b3r, p3[0], p3[1], da, db, wn3T)
    return out
